# Initial kernel scaffold; baseline (speedup 1.0000x reference)
import jax
import jax.numpy as jnp
from jax.experimental import pallas as pl


def kernel(context_features, context_labels, target_features):
    raise NotImplementedError("write your pallas kernel here")


# speedup vs baseline: 8.6004x; 8.6004x over previous
import functools

import jax
import jax.numpy as jnp
from jax.experimental import pallas as pl

NUM_CLASSES = 64
N_CTX = 320000
N_TGT = 32768
D = 128
SCALE = 50.0
EPS = 1e-30

CTX_BLOCK = 3200
TGT_BLOCK = 2048


def _seg_kernel(lab_ref, ctx_ref, sums_ref, counts_ref):
    @pl.when(pl.program_id(0) == 0)
    def _():
        sums_ref[...] = jnp.zeros_like(sums_ref)
        counts_ref[...] = jnp.zeros_like(counts_ref)

    lab = lab_ref[0, 0, :]
    onehot = (lab[:, None] == jax.lax.broadcasted_iota(jnp.int32, (1, NUM_CLASSES), 1)
              ).astype(jnp.float32)
    ctx = ctx_ref[...]
    sums_ref[...] += jax.lax.dot_general(
        onehot, ctx, (((0,), (0,)), ((), ())),
        preferred_element_type=jnp.float32)
    counts_ref[...] += jnp.sum(onehot, axis=0, keepdims=True)


def _logit_kernel(sums_ref, counts_ref, tgt_ref, logits_ref, means_ref):
    counts = counts_ref[...]
    means = sums_ref[...] / jnp.maximum(counts, 1.0).reshape(NUM_CLASSES, 1)
    means_ref[...] = means
    m_norm = jnp.maximum(
        jnp.sqrt(jnp.sum(means * means, axis=1, keepdims=True)), EPS)
    t = tgt_ref[...]
    t_norm = jnp.maximum(
        jnp.sqrt(jnp.sum(t * t, axis=1, keepdims=True)), EPS)
    dots = jax.lax.dot_general(
        t, means, (((1,), (1,)), ((), ())),
        preferred_element_type=jnp.float32)
    logits_ref[...] = dots * (SCALE / (t_norm * m_norm.reshape(1, NUM_CLASSES)))


@jax.jit
def kernel(context_features, context_labels, target_features):
    n_blocks = N_CTX // CTX_BLOCK
    labels3d = context_labels.reshape(n_blocks, 1, CTX_BLOCK)
    sums, counts = pl.pallas_call(
        _seg_kernel,
        grid=(n_blocks,),
        in_specs=[
            pl.BlockSpec((1, 1, CTX_BLOCK), lambda i: (i, 0, 0)),
            pl.BlockSpec((CTX_BLOCK, D), lambda i: (i, 0)),
        ],
        out_specs=[
            pl.BlockSpec((NUM_CLASSES, D), lambda i: (0, 0)),
            pl.BlockSpec((1, NUM_CLASSES), lambda i: (0, 0)),
        ],
        out_shape=[
            jax.ShapeDtypeStruct((NUM_CLASSES, D), jnp.float32),
            jax.ShapeDtypeStruct((1, NUM_CLASSES), jnp.float32),
        ],
    )(labels3d, context_features)

    t_blocks = N_TGT // TGT_BLOCK
    logits, means = pl.pallas_call(
        _logit_kernel,
        grid=(t_blocks,),
        in_specs=[
            pl.BlockSpec((NUM_CLASSES, D), lambda i: (0, 0)),
            pl.BlockSpec((1, NUM_CLASSES), lambda i: (0, 0)),
            pl.BlockSpec((TGT_BLOCK, D), lambda i: (i, 0)),
        ],
        out_specs=[
            pl.BlockSpec((TGT_BLOCK, NUM_CLASSES), lambda i: (i, 0)),
            pl.BlockSpec((NUM_CLASSES, D), lambda i: (0, 0)),
        ],
        out_shape=[
            jax.ShapeDtypeStruct((N_TGT, NUM_CLASSES), jnp.float32),
            jax.ShapeDtypeStruct((NUM_CLASSES, D), jnp.float32),
        ],
    )(sums, counts, target_features)
    return (logits, means)


# speedup vs baseline: 10.1294x; 1.1778x over previous
import functools

import jax
import jax.numpy as jnp
from jax.experimental import pallas as pl

NUM_CLASSES = 64
N_CTX = 320000
N_TGT = 32768
D = 128
SCALE = 50.0
EPS = 1e-30

CTX_BLOCK = 3200
TGT_BLOCK = 2048


def _seg_kernel(lab_ref, ctx_ref, sums_ref, counts_ref):
    @pl.when(pl.program_id(0) == 0)
    def _():
        sums_ref[...] = jnp.zeros_like(sums_ref)
        counts_ref[...] = jnp.zeros_like(counts_ref)

    lab = lab_ref[0, :, :]
    onehot_t = (jax.lax.broadcasted_iota(jnp.int32, (NUM_CLASSES, CTX_BLOCK), 0)
                == lab).astype(jnp.bfloat16)
    ctx = ctx_ref[...]
    sums_ref[...] += jax.lax.dot_general(
        onehot_t, ctx, (((1,), (0,)), ((), ())),
        preferred_element_type=jnp.float32)
    counts_ref[...] += jnp.sum(onehot_t.astype(jnp.float32), axis=1,
                               keepdims=True).reshape(1, NUM_CLASSES)


def _logit_kernel(sums_ref, counts_ref, tgt_ref, logits_ref, means_ref):
    counts = counts_ref[...]
    means = sums_ref[...] / jnp.maximum(counts, 1.0).reshape(NUM_CLASSES, 1)
    means_ref[...] = means
    m_norm = jnp.maximum(
        jnp.sqrt(jnp.sum(means * means, axis=1, keepdims=True)), EPS)
    means_scaled = means * (SCALE / m_norm)
    t = tgt_ref[...]
    ones_dc = jnp.ones((D, NUM_CLASSES), dtype=jnp.float32)
    nsq = jax.lax.dot_general(
        t * t, ones_dc, (((1,), (0,)), ((), ())),
        preferred_element_type=jnp.float32)
    inv_t = jax.lax.rsqrt(jnp.maximum(nsq, 1e-37))
    dots = jax.lax.dot_general(
        t, means_scaled, (((1,), (1,)), ((), ())),
        preferred_element_type=jnp.float32)
    logits_ref[...] = dots * inv_t


@jax.jit
def kernel(context_features, context_labels, target_features):
    n_blocks = N_CTX // CTX_BLOCK
    labels3d = context_labels.reshape(n_blocks, 1, CTX_BLOCK)
    sums, counts = pl.pallas_call(
        _seg_kernel,
        grid=(n_blocks,),
        in_specs=[
            pl.BlockSpec((1, 1, CTX_BLOCK), lambda i: (i, 0, 0)),
            pl.BlockSpec((CTX_BLOCK, D), lambda i: (i, 0)),
        ],
        out_specs=[
            pl.BlockSpec((NUM_CLASSES, D), lambda i: (0, 0)),
            pl.BlockSpec((1, NUM_CLASSES), lambda i: (0, 0)),
        ],
        out_shape=[
            jax.ShapeDtypeStruct((NUM_CLASSES, D), jnp.float32),
            jax.ShapeDtypeStruct((1, NUM_CLASSES), jnp.float32),
        ],
    )(labels3d, context_features)

    t_blocks = N_TGT // TGT_BLOCK
    logits, means = pl.pallas_call(
        _logit_kernel,
        grid=(t_blocks,),
        in_specs=[
            pl.BlockSpec((NUM_CLASSES, D), lambda i: (0, 0)),
            pl.BlockSpec((1, NUM_CLASSES), lambda i: (0, 0)),
            pl.BlockSpec((TGT_BLOCK, D), lambda i: (i, 0)),
        ],
        out_specs=[
            pl.BlockSpec((TGT_BLOCK, NUM_CLASSES), lambda i: (i, 0)),
            pl.BlockSpec((NUM_CLASSES, D), lambda i: (0, 0)),
        ],
        out_shape=[
            jax.ShapeDtypeStruct((N_TGT, NUM_CLASSES), jnp.float32),
            jax.ShapeDtypeStruct((NUM_CLASSES, D), jnp.float32),
        ],
    )(sums, counts, target_features)
    return (logits, means)
